# Initial kernel scaffold; baseline (speedup 1.0000x reference)
import jax
import jax.numpy as jnp
from jax.experimental import pallas as pl


def kernel(predictions, station_positions, station_runoffs):
    raise NotImplementedError("write your pallas kernel here")


# speedup vs baseline: 1.1644x; 1.1644x over previous
import functools

import jax
import jax.numpy as jnp
from jax import lax
from jax.experimental import pallas as pl
from jax.experimental.pallas import tpu as pltpu
from jax.experimental.pallas import tpu_sc as plsc

NC = 2
NS = 16
L = 16
NW = NC * NS

_OFFS = [(dy, dx) for dy in (-1, 0, 1) for dx in (-1, 0, 1)]


def _station_partials(pred_flat, px, py, run, *, n_stations, s_per_b, h, w,
                      per_w):
    groups = per_w // L
    mesh = plsc.VectorSubcoreMesh(
        core_axis_name="c", subcore_axis_name="s",
        num_cores=NC, num_subcores=NS)

    @functools.partial(
        pl.kernel,
        mesh=mesh,
        out_type=jax.ShapeDtypeStruct((NW, L), jnp.float32),
        scratch_types=[
            pltpu.VMEM((per_w,), jnp.int32),
            pltpu.VMEM((per_w,), jnp.int32),
            pltpu.VMEM((per_w,), jnp.float32),
            pltpu.VMEM((9, per_w), jnp.int32),
            pltpu.VMEM((9, per_w), jnp.float32),
            pltpu.VMEM((L,), jnp.float32),
            pltpu.SemaphoreType.DMA,
        ],
    )
    def body(pred_hbm, px_hbm, py_hbm, run_hbm, out_hbm,
             px_v, py_v, run_v, idx_v, vals_v, acc_v, sem):
        wid = lax.axis_index("s") * NC + lax.axis_index("c")
        base = wid * per_w

        pltpu.sync_copy(px_hbm.at[pl.ds(base, per_w)], px_v)
        pltpu.sync_copy(py_hbm.at[pl.ds(base, per_w)], py_v)
        pltpu.sync_copy(run_hbm.at[pl.ds(base, per_w)], run_v)

        hw_vec = jnp.zeros((L,), jnp.int32) + (h * w)
        zi = jnp.zeros((L,), jnp.int32)
        onesf = jnp.zeros((L,), jnp.float32) + 1.0
        zf = jnp.zeros((L,), jnp.float32)

        for k in range(groups):
            sl = pl.ds(k * L, L)
            px16 = px_v[sl]
            py16 = py_v[sl]
            g16 = base + k * L + lax.iota(jnp.int32, 16)
            bo16 = zi
            for m in range(1, n_stations // s_per_b):
                bo16 = bo16 + jnp.where(g16 >= m * s_per_b, hw_vec, zi)
            for j, (dy, dx) in enumerate(_OFFS):
                yc = jnp.clip(py16 + dy, 0, h - 1)
                xc = jnp.clip(px16 + dx, 0, w - 1)
                idx_v[j, sl] = bo16 + yc * w + xc

        copies = [
            pltpu.async_copy(pred_hbm.at[idx_v.at[j]], vals_v.at[j], sem)
            for j in range(9)
        ]
        for cp in copies:
            cp.wait()

        total16 = zf
        for k in range(groups):
            sl = pl.ds(k * L, L)
            px16 = px_v[sl]
            py16 = py_v[sl]
            run16 = run_v[sl]
            g16 = base + k * L + lax.iota(jnp.int32, 16)
            svalid = jnp.where(g16 < n_stations, onesf, zf)
            acc = zf
            cnt = zf
            for j, (dy, dx) in enumerate(_OFFS):
                y = py16 + dy
                x = px16 + dx
                ok = (y >= 0) & (y < h) & (x >= 0) & (x < w)
                okf = jnp.where(ok, onesf, zf)
                acc = acc + vals_v[j, sl] * okf
                cnt = cnt + okf
            d = acc / cnt - run16
            total16 = total16 + d * d * svalid

        acc_v[...] = total16
        pltpu.sync_copy(acc_v, out_hbm.at[wid])

    return body(pred_flat, px, py, run)


def _sum_partials(partials2d, *, n_stations):
    def body(x_ref, o_ref):
        o_ref[0, 0] = jnp.sum(x_ref[...]) * (1.0 / n_stations)

    return pl.pallas_call(
        body,
        out_shape=jax.ShapeDtypeStruct((1, 1), jnp.float32),
        out_specs=pl.BlockSpec(memory_space=pltpu.SMEM),
    )(partials2d)


def kernel(predictions, station_positions, station_runoffs):
    b, _, h, w = predictions.shape
    s = station_positions.shape[1]
    n_stations = b * s
    per_w = -(-n_stations // (NW * L)) * L
    n_pad = NW * per_w

    pred_flat = predictions.reshape(b * h * w)
    px = station_positions[..., 0].reshape(n_stations)
    py = station_positions[..., 1].reshape(n_stations)
    run = station_runoffs.reshape(n_stations)
    pad = n_pad - n_stations
    px = jnp.pad(px, (0, pad))
    py = jnp.pad(py, (0, pad))
    run = jnp.pad(run, (0, pad))

    partials = _station_partials(
        pred_flat, px, py, run, n_stations=n_stations, s_per_b=s,
        h=h, w=w, per_w=per_w)
    loss = _sum_partials(partials.reshape(4, 128), n_stations=n_stations)
    return loss[0, 0]


# speedup vs baseline: 1.1671x; 1.0023x over previous
import functools

import jax
import jax.numpy as jnp
from jax import lax
from jax.experimental import pallas as pl
from jax.experimental.pallas import tpu as pltpu
from jax.experimental.pallas import tpu_sc as plsc

NC = 2
NS = 16
L = 16
NW = NC * NS

_OFFS = [(dy, dx) for dy in (-1, 0, 1) for dx in (-1, 0, 1)]


def _station_partials(pred_flat, px, py, run, *, n_stations, s_per_b, h, w,
                      per_w):
    groups = per_w // L
    mesh = plsc.VectorSubcoreMesh(
        core_axis_name="c", subcore_axis_name="s",
        num_cores=NC, num_subcores=NS)

    @functools.partial(
        pl.kernel,
        mesh=mesh,
        out_type=jax.ShapeDtypeStruct((NW, L), jnp.float32),
        scratch_types=[
            pltpu.VMEM((per_w,), jnp.int32),
            pltpu.VMEM((per_w,), jnp.int32),
            pltpu.VMEM((per_w,), jnp.float32),
            pltpu.VMEM((9 * per_w,), jnp.int32),
            pltpu.VMEM((9 * per_w,), jnp.float32),
            pltpu.VMEM((L,), jnp.float32),
            pltpu.SemaphoreType.DMA,
        ],
    )
    def body(pred_hbm, px_hbm, py_hbm, run_hbm, out_hbm,
             px_v, py_v, run_v, idx_v, vals_v, acc_v, sem):
        wid = lax.axis_index("s") * NC + lax.axis_index("c")
        base = wid * per_w

        pltpu.sync_copy(px_hbm.at[pl.ds(base, per_w)], px_v)
        pltpu.sync_copy(py_hbm.at[pl.ds(base, per_w)], py_v)
        pltpu.sync_copy(run_hbm.at[pl.ds(base, per_w)], run_v)

        hw_vec = jnp.zeros((L,), jnp.int32) + (h * w)
        zi = jnp.zeros((L,), jnp.int32)
        onesf = jnp.zeros((L,), jnp.float32) + 1.0
        zf = jnp.zeros((L,), jnp.float32)

        for k in range(groups):
            sl = pl.ds(k * L, L)
            px16 = px_v[sl]
            py16 = py_v[sl]
            g16 = base + k * L + lax.iota(jnp.int32, 16)
            bo16 = zi
            for m in range(1, n_stations // s_per_b):
                bo16 = bo16 + jnp.where(g16 >= m * s_per_b, hw_vec, zi)
            for j, (dy, dx) in enumerate(_OFFS):
                yc = jnp.clip(py16 + dy, 0, h - 1)
                xc = jnp.clip(px16 + dx, 0, w - 1)
                idx_v[pl.ds(j * per_w + k * L, L)] = bo16 + yc * w + xc

        pltpu.async_copy(pred_hbm.at[idx_v], vals_v, sem).wait()

        total16 = zf
        for k in range(groups):
            sl = pl.ds(k * L, L)
            px16 = px_v[sl]
            py16 = py_v[sl]
            run16 = run_v[sl]
            g16 = base + k * L + lax.iota(jnp.int32, 16)
            svalid = jnp.where(g16 < n_stations, onesf, zf)
            acc = zf
            cnt = zf
            for j, (dy, dx) in enumerate(_OFFS):
                y = py16 + dy
                x = px16 + dx
                ok = (y >= 0) & (y < h) & (x >= 0) & (x < w)
                okf = jnp.where(ok, onesf, zf)
                acc = acc + vals_v[pl.ds(j * per_w + k * L, L)] * okf
                cnt = cnt + okf
            d = acc / cnt - run16
            total16 = total16 + d * d * svalid

        acc_v[...] = total16
        pltpu.sync_copy(acc_v, out_hbm.at[wid])

    return body(pred_flat, px, py, run)


def _sum_partials(partials2d, *, n_stations):
    def body(x_ref, o_ref):
        o_ref[0, 0] = jnp.sum(x_ref[...]) * (1.0 / n_stations)

    return pl.pallas_call(
        body,
        out_shape=jax.ShapeDtypeStruct((1, 1), jnp.float32),
        out_specs=pl.BlockSpec(memory_space=pltpu.SMEM),
    )(partials2d)


def kernel(predictions, station_positions, station_runoffs):
    b, _, h, w = predictions.shape
    s = station_positions.shape[1]
    n_stations = b * s
    per_w = -(-n_stations // (NW * L)) * L
    n_pad = NW * per_w

    pred_flat = predictions.reshape(b * h * w)
    px = station_positions[..., 0].reshape(n_stations)
    py = station_positions[..., 1].reshape(n_stations)
    run = station_runoffs.reshape(n_stations)
    pad = n_pad - n_stations
    px = jnp.pad(px, (0, pad))
    py = jnp.pad(py, (0, pad))
    run = jnp.pad(run, (0, pad))

    partials = _station_partials(
        pred_flat, px, py, run, n_stations=n_stations, s_per_b=s,
        h=h, w=w, per_w=per_w)
    loss = _sum_partials(partials.reshape(4, 128), n_stations=n_stations)
    return loss[0, 0]
